# baseline (speedup 1.0000x reference)
import jax
import jax.numpy as jnp
from jax.experimental import pallas as pl
from jax.experimental.pallas import tpu as pltpu

_LANES = 128
_CR = 8192
_K = 6
_WL = 2


def _ring_copy_kernel(idx_ref, a_ref, o_ref, rbufs, wbufs, *sems):
    rows = a_ref.shape[0]
    nch = rows // _CR
    rsem = sems[:_K]
    wsem = sems[_K:]
    idx = idx_ref[0]

    def rd(i):
        s = i % _K
        return pltpu.make_async_copy(
            a_ref.at[pl.ds(i * _CR, _CR), :], rbufs.at[s], rsem[s])

    def wr(i):
        s = i % _K
        return pltpu.make_async_copy(
            wbufs.at[s], o_ref.at[pl.ds(i * _CR, _CR), :], wsem[s])

    for i in range(min(_K, nch)):
        rd(i).start()

    waited = set()
    for i in range(nch):
        s = i % _K
        rd(i).wait()

        base = i * _CR * _LANES
        contains = jnp.logical_and(idx + 1 >= base,
                                   idx < base + _CR * _LANES)

        @pl.when(jnp.logical_not(contains))
        def _plain(s=s):
            wbufs[s] = rbufs[s]

        @pl.when(contains)
        def _masked(s=s, base=base):
            rr = jax.lax.broadcasted_iota(jnp.int32, (_CR, _LANES), 0)
            cc = jax.lax.broadcasted_iota(jnp.int32, (_CR, _LANES), 1)
            flat = base + rr * _LANES + cc
            mask = jnp.logical_or(flat == idx, flat == idx + 1)
            wbufs[s] = jnp.where(mask, jnp.float32(0), rbufs[s])

        wr(i).start()
        if i + _K < nch:
            rd(i + _K).start()
        j = i - _WL
        if j >= 0:
            wr(j).wait()
            waited.add(j)

    for i in range(nch):
        if i not in waited:
            wr(i).wait()


def kernel(a, idx):
    n = a.shape[0]
    rows = n // _LANES
    idx32 = idx.astype(jnp.int32)
    a2 = a.reshape(rows, _LANES)
    out = pl.pallas_call(
        _ring_copy_kernel,
        out_shape=jax.ShapeDtypeStruct((rows, _LANES), a.dtype),
        in_specs=[
            pl.BlockSpec(memory_space=pltpu.SMEM),
            pl.BlockSpec(memory_space=pltpu.MemorySpace.HBM),
        ],
        out_specs=pl.BlockSpec(memory_space=pltpu.MemorySpace.HBM),
        scratch_shapes=[
            pltpu.VMEM((_K, _CR, _LANES), jnp.float32),
            pltpu.VMEM((_K, _CR, _LANES), jnp.float32),
        ]
        + [pltpu.SemaphoreType.DMA] * (2 * _K),
    )(idx32, a2)
    return out.reshape(n)


# speedup vs baseline: 1.0074x; 1.0074x over previous
import jax
import jax.numpy as jnp
from jax.experimental import pallas as pl
from jax.experimental.pallas import tpu as pltpu

_LANES = 128
_CR = 16384
_K = 3
_WL = 2


def _ring_copy_kernel(idx_ref, a_ref, o_ref, rbufs, wbufs, *sems):
    rows = a_ref.shape[0]
    nch = rows // _CR
    rsem = sems[:_K]
    wsem = sems[_K:]
    idx = idx_ref[0]

    def rd(i):
        s = i % _K
        return pltpu.make_async_copy(
            a_ref.at[pl.ds(i * _CR, _CR), :], rbufs.at[s], rsem[s])

    def wr(i):
        s = i % _K
        return pltpu.make_async_copy(
            wbufs.at[s], o_ref.at[pl.ds(i * _CR, _CR), :], wsem[s])

    for i in range(min(_K, nch)):
        rd(i).start()

    waited = set()
    for i in range(nch):
        s = i % _K
        rd(i).wait()

        base = i * _CR * _LANES
        contains = jnp.logical_and(idx + 1 >= base,
                                   idx < base + _CR * _LANES)

        @pl.when(jnp.logical_not(contains))
        def _plain(s=s):
            wbufs[s] = rbufs[s]

        @pl.when(contains)
        def _masked(s=s, base=base):
            rr = jax.lax.broadcasted_iota(jnp.int32, (_CR, _LANES), 0)
            cc = jax.lax.broadcasted_iota(jnp.int32, (_CR, _LANES), 1)
            flat = base + rr * _LANES + cc
            mask = jnp.logical_or(flat == idx, flat == idx + 1)
            wbufs[s] = jnp.where(mask, jnp.float32(0), rbufs[s])

        wr(i).start()
        if i + _K < nch:
            rd(i + _K).start()
        j = i - _WL
        if j >= 0:
            wr(j).wait()
            waited.add(j)

    for i in range(nch):
        if i not in waited:
            wr(i).wait()


def kernel(a, idx):
    n = a.shape[0]
    rows = n // _LANES
    idx32 = idx.astype(jnp.int32)
    a2 = a.reshape(rows, _LANES)
    out = pl.pallas_call(
        _ring_copy_kernel,
        out_shape=jax.ShapeDtypeStruct((rows, _LANES), a.dtype),
        in_specs=[
            pl.BlockSpec(memory_space=pltpu.SMEM),
            pl.BlockSpec(memory_space=pltpu.MemorySpace.HBM),
        ],
        out_specs=pl.BlockSpec(memory_space=pltpu.MemorySpace.HBM),
        scratch_shapes=[
            pltpu.VMEM((_K, _CR, _LANES), jnp.float32),
            pltpu.VMEM((_K, _CR, _LANES), jnp.float32),
        ]
        + [pltpu.SemaphoreType.DMA] * (2 * _K),
    )(idx32, a2)
    return out.reshape(n)


# speedup vs baseline: 1.0076x; 1.0002x over previous
import jax
import jax.numpy as jnp
from jax.experimental import pallas as pl
from jax.experimental.pallas import tpu as pltpu

_LANES = 128
_CR = 16384
_K = 3
_WL = 1


def _ring_copy_kernel(idx_ref, a_ref, o_ref, rbufs, wbufs, *sems):
    rows = a_ref.shape[0]
    nch = rows // _CR
    rsem = sems[:_K]
    wsem = sems[_K:]
    idx = idx_ref[0]

    def rd(i):
        s = i % _K
        return pltpu.make_async_copy(
            a_ref.at[pl.ds(i * _CR, _CR), :], rbufs.at[s], rsem[s])

    def wr(i):
        s = i % _K
        return pltpu.make_async_copy(
            wbufs.at[s], o_ref.at[pl.ds(i * _CR, _CR), :], wsem[s])

    for i in range(min(_K, nch)):
        rd(i).start()

    waited = set()
    for i in range(nch):
        s = i % _K
        rd(i).wait()

        base = i * _CR * _LANES
        contains = jnp.logical_and(idx + 1 >= base,
                                   idx < base + _CR * _LANES)

        @pl.when(jnp.logical_not(contains))
        def _plain(s=s):
            wbufs[s] = rbufs[s]

        @pl.when(contains)
        def _masked(s=s, base=base):
            rr = jax.lax.broadcasted_iota(jnp.int32, (_CR, _LANES), 0)
            cc = jax.lax.broadcasted_iota(jnp.int32, (_CR, _LANES), 1)
            flat = base + rr * _LANES + cc
            mask = jnp.logical_or(flat == idx, flat == idx + 1)
            wbufs[s] = jnp.where(mask, jnp.float32(0), rbufs[s])

        wr(i).start()
        if i + _K < nch:
            rd(i + _K).start()
        j = i - _WL
        if j >= 0:
            wr(j).wait()
            waited.add(j)

    for i in range(nch):
        if i not in waited:
            wr(i).wait()


def kernel(a, idx):
    n = a.shape[0]
    rows = n // _LANES
    idx32 = idx.astype(jnp.int32)
    a2 = a.reshape(rows, _LANES)
    out = pl.pallas_call(
        _ring_copy_kernel,
        out_shape=jax.ShapeDtypeStruct((rows, _LANES), a.dtype),
        in_specs=[
            pl.BlockSpec(memory_space=pltpu.SMEM),
            pl.BlockSpec(memory_space=pltpu.MemorySpace.HBM),
        ],
        out_specs=pl.BlockSpec(memory_space=pltpu.MemorySpace.HBM),
        scratch_shapes=[
            pltpu.VMEM((_K, _CR, _LANES), jnp.float32),
            pltpu.VMEM((_K, _CR, _LANES), jnp.float32),
        ]
        + [pltpu.SemaphoreType.DMA] * (2 * _K),
    )(idx32, a2)
    return out.reshape(n)


# speedup vs baseline: 1.0084x; 1.0009x over previous
import jax
import jax.numpy as jnp
from jax.experimental import pallas as pl
from jax.experimental.pallas import tpu as pltpu

_LANES = 128
_CR = 16384
_K = 2
_WL = 1


def _ring_copy_kernel(idx_ref, a_ref, o_ref, rbufs, wbufs, *sems):
    rows = a_ref.shape[0]
    nch = rows // _CR
    rsem = sems[:_K]
    wsem = sems[_K:]
    idx = idx_ref[0]

    def rd(i):
        s = i % _K
        return pltpu.make_async_copy(
            a_ref.at[pl.ds(i * _CR, _CR), :], rbufs.at[s], rsem[s])

    def wr(i):
        s = i % _K
        return pltpu.make_async_copy(
            wbufs.at[s], o_ref.at[pl.ds(i * _CR, _CR), :], wsem[s])

    for i in range(min(_K, nch)):
        rd(i).start()

    waited = set()
    for i in range(nch):
        s = i % _K
        rd(i).wait()

        base = i * _CR * _LANES
        contains = jnp.logical_and(idx + 1 >= base,
                                   idx < base + _CR * _LANES)

        @pl.when(jnp.logical_not(contains))
        def _plain(s=s):
            wbufs[s] = rbufs[s]

        @pl.when(contains)
        def _masked(s=s, base=base):
            rr = jax.lax.broadcasted_iota(jnp.int32, (_CR, _LANES), 0)
            cc = jax.lax.broadcasted_iota(jnp.int32, (_CR, _LANES), 1)
            flat = base + rr * _LANES + cc
            mask = jnp.logical_or(flat == idx, flat == idx + 1)
            wbufs[s] = jnp.where(mask, jnp.float32(0), rbufs[s])

        wr(i).start()
        if i + _K < nch:
            rd(i + _K).start()
        j = i - _WL
        if j >= 0:
            wr(j).wait()
            waited.add(j)

    for i in range(nch):
        if i not in waited:
            wr(i).wait()


def kernel(a, idx):
    n = a.shape[0]
    rows = n // _LANES
    idx32 = idx.astype(jnp.int32)
    a2 = a.reshape(rows, _LANES)
    out = pl.pallas_call(
        _ring_copy_kernel,
        out_shape=jax.ShapeDtypeStruct((rows, _LANES), a.dtype),
        in_specs=[
            pl.BlockSpec(memory_space=pltpu.SMEM),
            pl.BlockSpec(memory_space=pltpu.MemorySpace.HBM),
        ],
        out_specs=pl.BlockSpec(memory_space=pltpu.MemorySpace.HBM),
        scratch_shapes=[
            pltpu.VMEM((_K, _CR, _LANES), jnp.float32),
            pltpu.VMEM((_K, _CR, _LANES), jnp.float32),
        ]
        + [pltpu.SemaphoreType.DMA] * (2 * _K),
    )(idx32, a2)
    return out.reshape(n)
